# baseline (speedup 1.0000x reference)
import functools

import jax
import jax.numpy as jnp
from jax import lax
from jax.experimental import pallas as pl
from jax.experimental.pallas import tpu as pltpu
from jax.experimental.pallas import tpu_sc as plsc

N = 10000
D = 128
NW = 32
LANES = 16
R = 320
NPAD = NW * R
C = 128
STAGE_E = 11264
STAGE_C = STAGE_E // C
BLK_E = 1280
SBUF = 1152
CW = 128
UBW = 384
FLUSH = 1024
NEG_INF = float("-inf")


def _mesh():
    return plsc.VectorSubcoreMesh(core_axis_name="c", subcore_axis_name="s")


def _wid():
    return lax.axis_index("s") * 2 + lax.axis_index("c")


def _round_up(v, m):
    return (v + m - 1) // m * m


@functools.lru_cache(maxsize=None)
def _partition_fn(E):
    assert E % BLK_E == 0 and BLK_E % LANES == 0
    CAP = _round_up(E, STAGE_E)
    nblk = E // BLK_E
    npair = nblk // 2
    assert nblk % 2 == 0

    @functools.partial(
        pl.kernel,
        out_type=(
            jax.ShapeDtypeStruct((NW * CAP,), jnp.int32),
            jax.ShapeDtypeStruct((NW * CAP,), jnp.float32),
            jax.ShapeDtypeStruct((NW * CAP,), jnp.int32),
            jax.ShapeDtypeStruct((NW * CAP,), jnp.float32),
            jax.ShapeDtypeStruct((NW * CW,), jnp.int32),
        ),
        mesh=_mesh(),
        compiler_params=pltpu.CompilerParams(needs_layout_passes=False, disable_bounds_checks=True),
        scratch_types=[
            pltpu.VMEM((2, BLK_E), jnp.int32),
            pltpu.VMEM((2, BLK_E), jnp.int32),
            pltpu.VMEM((2, BLK_E), jnp.float32),
            pltpu.VMEM((NPAD,), jnp.float32),
            pltpu.VMEM((SBUF,), jnp.int32),
            pltpu.VMEM((SBUF,), jnp.float32),
            pltpu.VMEM((SBUF,), jnp.int32),
            pltpu.VMEM((SBUF,), jnp.float32),
            pltpu.VMEM((CW,), jnp.int32),
            pltpu.SemaphoreType.DMA,
            pltpu.SemaphoreType.DMA,
        ],
    )
    def part(src_h, dst_h, w_h, prv_h, srcL, wL, dlL, pwL, cnts,
             sblk, dblk, wblk, prv_v, sst, wst, dst_st, pwst, cv, semA, semB):
        wid = _wid()
        base = wid * R
        lbase = wid * CAP

        pltpu.sync_copy(prv_h, prv_v)

        def issue(blk, slot, sem):
            off = pl.multiple_of(blk * BLK_E, 8)
            pltpu.async_copy(src_h.at[pl.ds(off, BLK_E)], sblk.at[slot], sem)
            pltpu.async_copy(dst_h.at[pl.ds(off, BLK_E)], dblk.at[slot], sem)
            pltpu.async_copy(w_h.at[pl.ds(off, BLK_E)], wblk.at[slot], sem)

        def wait(slot, sem):
            pltpu.make_async_copy(src_h.at[pl.ds(0, BLK_E)], sblk.at[slot], sem).wait()
            pltpu.make_async_copy(dst_h.at[pl.ds(0, BLK_E)], dblk.at[slot], sem).wait()
            pltpu.make_async_copy(w_h.at[pl.ds(0, BLK_E)], wblk.at[slot], sem).wait()

        issue(0, 0, semA)
        issue(1, 1, semB)

        def scan_block(slot, pending, written):
            def quad(q, carry):
                pending, written = carry
                svs, dvs, wvs, ms, pws, cs = [], [], [], [], [], []
                for u in range(8):
                    off = q * (8 * LANES) + u * LANES
                    sv = sblk[slot, pl.ds(off, LANES)]
                    dv = dblk[slot, pl.ds(off, LANES)]
                    wv = wblk[slot, pl.ds(off, LANES)]
                    m = (dv >= base) & (dv < base + R)
                    svs.append(sv)
                    dvs.append(dv - base)
                    wvs.append(wv)
                    ms.append(m)
                    pws.append(plsc.load_gather(prv_v, [sv]) * wv)
                    cs.append(plsc.all_reduce_population_count(m)[0])
                offs = [pending]
                for u in range(7):
                    offs.append(offs[u] + cs[u])
                for u in range(8):
                    plsc.store_compressed(sst.at[pl.ds(offs[u], LANES)], svs[u],
                                          mask=ms[u])
                    plsc.store_compressed(wst.at[pl.ds(offs[u], LANES)], wvs[u],
                                          mask=ms[u])
                    plsc.store_compressed(dst_st.at[pl.ds(offs[u], LANES)],
                                          dvs[u], mask=ms[u])
                    plsc.store_compressed(pwst.at[pl.ds(offs[u], LANES)], pws[u],
                                          mask=ms[u])
                pending = offs[7] + cs[7]
                do_flush = pending >= FLUSH

                @pl.when(do_flush)
                def _():
                    wo = pl.multiple_of(lbase + written, 8)
                    pltpu.sync_copy(sst.at[pl.ds(0, FLUSH)],
                                    srcL.at[pl.ds(wo, FLUSH)])
                    pltpu.sync_copy(wst.at[pl.ds(0, FLUSH)],
                                    wL.at[pl.ds(wo, FLUSH)])
                    pltpu.sync_copy(dst_st.at[pl.ds(0, FLUSH)],
                                    dlL.at[pl.ds(wo, FLUSH)])
                    pltpu.sync_copy(pwst.at[pl.ds(0, FLUSH)],
                                    pwL.at[pl.ds(wo, FLUSH)])
                    for u in range(8):
                        mo = u * LANES
                        sst[pl.ds(mo, LANES)] = sst[pl.ds(FLUSH + mo, LANES)]
                        wst[pl.ds(mo, LANES)] = wst[pl.ds(FLUSH + mo, LANES)]
                        dst_st[pl.ds(mo, LANES)] = dst_st[pl.ds(FLUSH + mo, LANES)]
                        pwst[pl.ds(mo, LANES)] = pwst[pl.ds(FLUSH + mo, LANES)]

                pending = jnp.where(do_flush, pending - FLUSH, pending)
                written = jnp.where(do_flush, written + FLUSH, written)
                return pending, written

            return lax.fori_loop(0, BLK_E // (8 * LANES), quad, (pending, written))

        def pair(p, carry):
            pending, written = carry
            wait(0, semA)
            pending, written = scan_block(0, pending, written)

            @pl.when(2 * p + 2 < nblk)
            def _():
                issue(2 * p + 2, 0, semA)

            wait(1, semB)
            pending, written = scan_block(1, pending, written)

            @pl.when(2 * p + 3 < nblk)
            def _():
                issue(2 * p + 3, 1, semB)

            return pending, written

        pending, written = lax.fori_loop(0, npair, pair, (0, 0))

        pad = (C - pending % C) % C
        zi = jnp.zeros((LANES,), jnp.int32)
        zf = jnp.zeros((LANES,), jnp.float32)
        ri = jnp.full((LANES,), R, jnp.int32)

        def padv(k, _):
            off = pending + k * LANES
            sst[pl.ds(off, LANES)] = zi
            wst[pl.ds(off, LANES)] = zf
            dst_st[pl.ds(off, LANES)] = ri
            pwst[pl.ds(off, LANES)] = zf
            return 0

        lax.fori_loop(0, (pad + LANES - 1) // LANES, padv, 0)
        padded = pending + pad

        def fflush(k, _):
            o = pl.multiple_of(k * C, 8)
            wo = pl.multiple_of(lbase + written + k * C, 8)
            pltpu.sync_copy(sst.at[pl.ds(o, C)], srcL.at[pl.ds(wo, C)])
            pltpu.sync_copy(wst.at[pl.ds(o, C)], wL.at[pl.ds(wo, C)])
            pltpu.sync_copy(dst_st.at[pl.ds(o, C)], dlL.at[pl.ds(wo, C)])
            pltpu.sync_copy(pwst.at[pl.ds(o, C)], pwL.at[pl.ds(wo, C)])
            return 0

        lax.fori_loop(0, padded // C, fflush, 0)
        nchunks = (written + padded) // C
        lane = lax.broadcasted_iota(jnp.int32, (LANES,), 0)
        for i in range(CW // LANES):
            cv[pl.ds(i * LANES, LANES)] = jnp.where(lane == 0, nchunks, 0)
        pltpu.sync_copy(cv, cnts.at[pl.ds(pl.multiple_of(wid * CW, 8), CW)])

    return part


@functools.lru_cache(maxsize=None)
def _layer_fn(E, with_ub):
    CAP = _round_up(E, STAGE_E)
    out_type = [jax.ShapeDtypeStruct((NPAD, D), jnp.float32)]
    if with_ub:
        out_type.append(jax.ShapeDtypeStruct((NW * UBW,), jnp.float32))
    QW = 32
    NQF = (R + 8) * QW
    NB = 2 if with_ub else 3
    scratch = [
        pltpu.VMEM((NB, C, D), jnp.float32),
        pltpu.VMEM((STAGE_E,), jnp.int32),
        pltpu.VMEM((STAGE_E,), jnp.float32),
        pltpu.VMEM((STAGE_E,), jnp.int32),
        pltpu.VMEM((NQF,), jnp.float32),
        pltpu.VMEM((NQF,), jnp.float32),
        pltpu.VMEM((NQF,), jnp.float32),
        pltpu.VMEM((NQF,), jnp.float32),
        pltpu.VMEM((CW,), jnp.int32),
        pltpu.SemaphoreType.DMA((NB,)),
    ]
    if with_ub:
        scratch.insert(4, pltpu.VMEM((STAGE_E,), jnp.float32))
        scratch.insert(9, pltpu.VMEM(((R + 8) * LANES,), jnp.float32))
        scratch.insert(10, pltpu.VMEM((UBW,), jnp.float32))

    @functools.partial(
        pl.kernel,
        out_type=tuple(out_type) if with_ub else out_type[0],
        mesh=_mesh(),
        compiler_params=pltpu.CompilerParams(needs_layout_passes=False, disable_bounds_checks=True),
        scratch_types=scratch,
    )
    def layer(hp_h, srcL, wL, dlL, pwL, cnts, *rest):
        if with_ub:
            (neigh_o, ub_o, rows, sstg, wstg, dstg, pwstg, nq0, nq1, nq2, nq3,
             ubL, ubc, cv, gsem) = rest
        else:
            (neigh_o, rows, sstg, wstg, dstg, nq0, nq1, nq2, nq3,
             cv, gsem) = rest
            ub_o = ubL = ubc = pwstg = None
        nqs = (nq0, nq1, nq2, nq3)
        wid = _wid()
        lbase = wid * CAP

        pltpu.sync_copy(cnts.at[pl.ds(pl.multiple_of(wid * CW, 8), CW)], cv)
        nc = cv[pl.ds(0, LANES)][0]

        minf = jnp.full((LANES,), NEG_INF, jnp.float32)
        zf = jnp.zeros((LANES,), jnp.float32)

        def initrow(i, _):
            o = i * LANES
            for q in range(4):
                nqs[q][pl.ds(o, LANES)] = minf
            if with_ub:
                ubL[pl.ds(o, LANES)] = zf
            return 0

        lax.fori_loop(0, NQF // LANES, initrow, 0)

        nstages = (nc + STAGE_C - 1) // STAGE_C

        def stage(s, _):
            eb = pl.multiple_of(lbase + s * STAGE_E, 8)
            pltpu.sync_copy(srcL.at[pl.ds(eb, STAGE_E)], sstg)
            pltpu.sync_copy(wL.at[pl.ds(eb, STAGE_E)], wstg)
            pltpu.sync_copy(dlL.at[pl.ds(eb, STAGE_E)], dstg)
            if with_ub:
                pltpu.sync_copy(pwL.at[pl.ds(eb, STAGE_E)], pwstg)
            tc = jnp.minimum(STAGE_C, nc - s * STAGE_C)

            pltpu.async_copy(hp_h.at[sstg.at[pl.ds(pl.multiple_of(0, 8), C)]], rows.at[0], gsem.at[0])
            if NB > 2:
                @pl.when(1 < tc)
                def _():
                    pltpu.async_copy(
                        hp_h.at[sstg.at[pl.ds(pl.multiple_of(C, 8), C)]],
                        rows.at[1], gsem.at[1])

            def chunk(k, _):
                kk = k % NB
                nk = (k + NB - 1) % NB

                @pl.when(k + NB - 1 < tc)
                def _():
                    pltpu.async_copy(hp_h.at[sstg.at[pl.ds(pl.multiple_of((k + NB - 1) * C, 8), C)]],
                                     rows.at[nk], gsem.at[nk])

                pltpu.make_async_copy(hp_h.at[sstg.at[pl.ds(pl.multiple_of(k * C, 8), C)]],
                                      rows.at[kk], gsem.at[kk]).wait()

                def group(g, _):
                    eg = k * C + g * LANES
                    dv = dstg[pl.ds(eg, LANES)] * QW
                    ojs = [dv[j] for j in range(LANES)]
                    for j in range(LANES):
                        ev = jnp.full((LANES,), eg + j, jnp.int32)
                        wj = plsc.load_gather(wstg, [ev])
                        o = ojs[j]
                        jj = g * LANES + j
                        blocks = [(q, u) for q in range(4)
                                  for u in range(QW // LANES)]
                        segs = [rows[kk, jj, pl.ds(q * QW + u * LANES, LANES)]
                                for q, u in blocks]
                        curs = [nqs[q][pl.ds(o + u * LANES, LANES)]
                                for q, u in blocks]
                        news = [jnp.maximum(c, sg * wj)
                                for c, sg in zip(curs, segs)]
                        for (q, u), nv in zip(blocks, news):
                            nqs[q][pl.ds(o + u * LANES, LANES)] = nv
                        if with_ub:
                            pwj = plsc.load_gather(pwstg, [ev])
                            ubL[pl.ds(o // 2, LANES)] = \
                                ubL[pl.ds(o // 2, LANES)] + pwj
                    return 0

                lax.fori_loop(0, C // LANES, group, 0)
                return 0

            lax.fori_loop(0, tc, chunk, 0)
            return 0

        lax.fori_loop(0, nstages, stage, 0)

        rowbase = pl.multiple_of(wid * R, 8)
        for b, nrow in ((0, C), (1, C), (2, R - 2 * C)):
            def mrow(r, _):
                for q in range(4):
                    for u in range(QW // LANES):
                        f = q * QW + u * LANES
                        rows[0, r, pl.ds(f, LANES)] = \
                            nqs[q][pl.ds((b * C + r) * QW + u * LANES, LANES)]
                return 0

            lax.fori_loop(0, nrow, mrow, 0)
            pltpu.sync_copy(
                rows.at[0, pl.ds(0, nrow)],
                neigh_o.at[pl.ds(rowbase + b * C, nrow)])
        if with_ub:
            lane = lax.broadcasted_iota(jnp.int32, (LANES,), 0)
            zl = jnp.zeros((LANES,), jnp.int32)

            def compact(g, _):
                vals = plsc.load_gather(ubL, [(g * LANES + lane) * LANES + zl])
                ubc[pl.ds(g * LANES, LANES)] = vals
                return 0

            lax.fori_loop(0, R // LANES, compact, 0)
            pltpu.sync_copy(ubc, ub_o.at[pl.ds(pl.multiple_of(wid * UBW, 8), UBW)])

    return layer


def _dotT(a, b):
    return lax.dot_general(a, b, (((1,), (1,)), ((), ())),
                           preferred_element_type=jnp.float32)


TB = 2000


def _rowblk(width):
    return pl.BlockSpec((TB, width), lambda g: (g, 0))


def _full(shape):
    return pl.BlockSpec(shape, lambda g: tuple(0 for _ in shape))


def _fix_neigh(v):
    return jnp.where(v == NEG_INF, 0.0, v)


def _tc_pre_body(x_ref, wp_ref, bp_ref, o_ref):
    o_ref[...] = _dotT(x_ref[...], wp_ref[...]) + bp_ref[...]


def _tc_pre(x, wp, bp):
    return pl.pallas_call(
        _tc_pre_body,
        grid=(N // TB,),
        in_specs=[_rowblk(D), _full((D, D)), _full((1, D))],
        out_specs=_rowblk(D),
        out_shape=jax.ShapeDtypeStruct((N, D), jnp.float32),
    )(x, wp, bp.reshape(1, D))


def _tc_mid_body(h_ref, n_ref, wn_ref, bn_ref, wp_ref, bp_ref, h1_ref, hp_ref):
    h = h_ref[...]
    nf = _fix_neigh(n_ref[...])
    wn = wn_ref[...]
    h1 = _dotT(h, wn[:, :D]) + _dotT(nf, wn[:, D:]) + bn_ref[...]
    h1 = jnp.maximum(h1, 0.0)
    h1_ref[...] = h1
    hp_ref[...] = _dotT(h1, wp_ref[...]) + bp_ref[...]


def _tc_mid(h, neigh_raw, wn, bn, wp, bp):
    return pl.pallas_call(
        _tc_mid_body,
        grid=(N // TB,),
        in_specs=[_rowblk(D), _rowblk(D), _full((D, 2 * D)), _full((1, D)),
                  _full((D, D)), _full((1, D))],
        out_specs=(_rowblk(D), _rowblk(D)),
        out_shape=(jax.ShapeDtypeStruct((N, D), jnp.float32),
                   jax.ShapeDtypeStruct((N, D), jnp.float32)),
    )(h, neigh_raw, wn, bn.reshape(1, D), wp, bp.reshape(1, D))


def _tc_fin_body(h_ref, n_ref, x_ref, dub_ref, wn_ref, bn_ref, o_ref):
    h = h_ref[...]
    nf = _fix_neigh(n_ref[...])
    wn = wn_ref[...]
    s = _dotT(h, wn[:, :D]) + _dotT(nf, wn[:, D:]) + bn_ref[...]
    h3 = jnp.maximum(s, 0.0)
    now = x_ref[...][:, D - 1:D]
    ub = jnp.clip(now + dub_ref[...], 0.0, 1.0)
    o_ref[...] = jnp.minimum(now + h3, ub)


def _tc_fin(h2, neigh_raw, x, dub, wn, bn):
    return pl.pallas_call(
        _tc_fin_body,
        grid=(N // TB,),
        in_specs=[_rowblk(D), _rowblk(D), _rowblk(D), _rowblk(1),
                  _full((1, 2 * D)), _full((1, 1))],
        out_specs=_rowblk(1),
        out_shape=jax.ShapeDtypeStruct((N, 1), jnp.float32),
    )(h2, neigh_raw, x, dub, wn, bn.reshape(1, 1))


def kernel(x, edge_index, edge_weight, params):
    E = edge_index.shape[1]
    src = edge_index[0].astype(jnp.int32)
    dst = edge_index[1].astype(jnp.int32)
    w = edge_weight.astype(jnp.float32)
    prv = jnp.pad(x[:, D - 2], (0, NPAD - N))

    srcL, wL, dlL, pwL, cnts = _partition_fn(E)(src, dst, w, prv)

    hp0 = _tc_pre(x, params["W_pool_0"], params["b_pool_0"])
    neigh0, ub = _layer_fn(E, True)(hp0, srcL, wL, dlL, pwL, cnts)
    h1, hp1 = _tc_mid(x, neigh0, params["W_neigh_0"], params["b_neigh_0"],
                      params["W_pool_1"], params["b_pool_1"])
    neigh1 = _layer_fn(E, False)(hp1, srcL, wL, dlL, pwL, cnts)
    h2, hp2 = _tc_mid(h1, neigh1, params["W_neigh_1"], params["b_neigh_1"],
                      params["W_pool_2"], params["b_pool_2"])
    neigh2 = _layer_fn(E, False)(hp2, srcL, wL, dlL, pwL, cnts)
    dub = ub.reshape(NW, UBW)[:, :R].reshape(NPAD)[:N].reshape(N, 1)
    out = _tc_fin(h2, neigh2, x, dub, params["W_neigh_2"], params["b_neigh_2"])
    return out[:, 0]


# speedup vs baseline: 1.0341x; 1.0341x over previous
import functools

import jax
import jax.numpy as jnp
from jax import lax
from jax.experimental import pallas as pl
from jax.experimental.pallas import tpu as pltpu
from jax.experimental.pallas import tpu_sc as plsc

N = 10000
D = 128
NW = 32
LANES = 16
R = 320
NPAD = NW * R
C = 128
STAGE_E = 11264
STAGE_C = STAGE_E // C
BLK_E = 1280
SBUF = 1152
CW = 128
UBW = 384
FLUSH = 1024
PACKB = 14
PACK = 2 ** PACKB
NEG_INF = float("-inf")


def _mesh():
    return plsc.VectorSubcoreMesh(core_axis_name="c", subcore_axis_name="s")


def _wid():
    return lax.axis_index("s") * 2 + lax.axis_index("c")


def _round_up(v, m):
    return (v + m - 1) // m * m


@functools.lru_cache(maxsize=None)
def _partition_fn(E):
    assert E % BLK_E == 0 and BLK_E % LANES == 0
    CAP = _round_up(E, STAGE_E)
    nblk = E // BLK_E
    npair = nblk // 2
    assert nblk % 2 == 0

    @functools.partial(
        pl.kernel,
        out_type=(
            jax.ShapeDtypeStruct((NW * CAP,), jnp.int32),
            jax.ShapeDtypeStruct((NW * CAP,), jnp.float32),
            jax.ShapeDtypeStruct((NW * CAP,), jnp.int32),
            jax.ShapeDtypeStruct((NW * CAP,), jnp.float32),
            jax.ShapeDtypeStruct((NW * CW,), jnp.int32),
        ),
        mesh=_mesh(),
        compiler_params=pltpu.CompilerParams(needs_layout_passes=False, disable_bounds_checks=True),
        scratch_types=[
            pltpu.VMEM((2, BLK_E), jnp.int32),
            pltpu.VMEM((2, BLK_E), jnp.float32),
            pltpu.VMEM((NPAD,), jnp.float32),
            pltpu.VMEM((SBUF,), jnp.int32),
            pltpu.VMEM((SBUF,), jnp.float32),
            pltpu.VMEM((SBUF,), jnp.int32),
            pltpu.VMEM((SBUF,), jnp.float32),
            pltpu.VMEM((CW,), jnp.int32),
            pltpu.SemaphoreType.DMA,
            pltpu.SemaphoreType.DMA,
        ],
    )
    def part(ep_h, w_h, prv_h, srcL, wL, dlL, pwL, cnts,
             eblk, wblk, prv_v, sst, wst, dst_st, pwst, cv, semA, semB):
        wid = _wid()
        base = wid * R
        lbase = wid * CAP

        pltpu.sync_copy(prv_h, prv_v)

        def issue(blk, slot, sem):
            off = pl.multiple_of(blk * BLK_E, 8)
            pltpu.async_copy(ep_h.at[pl.ds(off, BLK_E)], eblk.at[slot], sem)
            pltpu.async_copy(w_h.at[pl.ds(off, BLK_E)], wblk.at[slot], sem)

        def wait(slot, sem):
            pltpu.make_async_copy(ep_h.at[pl.ds(0, BLK_E)], eblk.at[slot], sem).wait()
            pltpu.make_async_copy(w_h.at[pl.ds(0, BLK_E)], wblk.at[slot], sem).wait()

        issue(0, 0, semA)
        issue(1, 1, semB)

        def scan_block(slot, pending, written):
            def quad(q, carry):
                pending, written = carry
                svs, dvs, wvs, ms, pws, cs = [], [], [], [], [], []
                for u in range(8):
                    off = q * (8 * LANES) + u * LANES
                    ev = eblk[slot, pl.ds(off, LANES)]
                    sv = ev & (PACK - 1)
                    dv = lax.shift_right_logical(ev, PACKB)
                    wv = wblk[slot, pl.ds(off, LANES)]
                    m = (dv >= base) & (dv < base + R)
                    svs.append(sv)
                    dvs.append(dv - base)
                    wvs.append(wv)
                    ms.append(m)
                    pws.append(plsc.load_gather(prv_v, [sv]) * wv)
                    cs.append(plsc.all_reduce_population_count(m)[0])
                offs = [pending]
                for u in range(7):
                    offs.append(offs[u] + cs[u])
                for u in range(8):
                    plsc.store_compressed(sst.at[pl.ds(offs[u], LANES)], svs[u],
                                          mask=ms[u])
                    plsc.store_compressed(wst.at[pl.ds(offs[u], LANES)], wvs[u],
                                          mask=ms[u])
                    plsc.store_compressed(dst_st.at[pl.ds(offs[u], LANES)],
                                          dvs[u], mask=ms[u])
                    plsc.store_compressed(pwst.at[pl.ds(offs[u], LANES)], pws[u],
                                          mask=ms[u])
                pending = offs[7] + cs[7]
                do_flush = pending >= FLUSH

                @pl.when(do_flush)
                def _():
                    wo = pl.multiple_of(lbase + written, 8)
                    pltpu.sync_copy(sst.at[pl.ds(0, FLUSH)],
                                    srcL.at[pl.ds(wo, FLUSH)])
                    pltpu.sync_copy(wst.at[pl.ds(0, FLUSH)],
                                    wL.at[pl.ds(wo, FLUSH)])
                    pltpu.sync_copy(dst_st.at[pl.ds(0, FLUSH)],
                                    dlL.at[pl.ds(wo, FLUSH)])
                    pltpu.sync_copy(pwst.at[pl.ds(0, FLUSH)],
                                    pwL.at[pl.ds(wo, FLUSH)])
                    for u in range(8):
                        mo = u * LANES
                        sst[pl.ds(mo, LANES)] = sst[pl.ds(FLUSH + mo, LANES)]
                        wst[pl.ds(mo, LANES)] = wst[pl.ds(FLUSH + mo, LANES)]
                        dst_st[pl.ds(mo, LANES)] = dst_st[pl.ds(FLUSH + mo, LANES)]
                        pwst[pl.ds(mo, LANES)] = pwst[pl.ds(FLUSH + mo, LANES)]

                pending = jnp.where(do_flush, pending - FLUSH, pending)
                written = jnp.where(do_flush, written + FLUSH, written)
                return pending, written

            return lax.fori_loop(0, BLK_E // (8 * LANES), quad, (pending, written))

        def pair(p, carry):
            pending, written = carry
            wait(0, semA)
            pending, written = scan_block(0, pending, written)

            @pl.when(2 * p + 2 < nblk)
            def _():
                issue(2 * p + 2, 0, semA)

            wait(1, semB)
            pending, written = scan_block(1, pending, written)

            @pl.when(2 * p + 3 < nblk)
            def _():
                issue(2 * p + 3, 1, semB)

            return pending, written

        pending, written = lax.fori_loop(0, npair, pair, (0, 0))

        pad = (C - pending % C) % C
        zi = jnp.zeros((LANES,), jnp.int32)
        zf = jnp.zeros((LANES,), jnp.float32)
        ri = jnp.full((LANES,), R, jnp.int32)

        def padv(k, _):
            off = pending + k * LANES
            sst[pl.ds(off, LANES)] = zi
            wst[pl.ds(off, LANES)] = zf
            dst_st[pl.ds(off, LANES)] = ri
            pwst[pl.ds(off, LANES)] = zf
            return 0

        lax.fori_loop(0, (pad + LANES - 1) // LANES, padv, 0)
        padded = pending + pad

        def fflush(k, _):
            o = pl.multiple_of(k * C, 8)
            wo = pl.multiple_of(lbase + written + k * C, 8)
            pltpu.sync_copy(sst.at[pl.ds(o, C)], srcL.at[pl.ds(wo, C)])
            pltpu.sync_copy(wst.at[pl.ds(o, C)], wL.at[pl.ds(wo, C)])
            pltpu.sync_copy(dst_st.at[pl.ds(o, C)], dlL.at[pl.ds(wo, C)])
            pltpu.sync_copy(pwst.at[pl.ds(o, C)], pwL.at[pl.ds(wo, C)])
            return 0

        lax.fori_loop(0, padded // C, fflush, 0)
        nchunks = (written + padded) // C
        lane = lax.broadcasted_iota(jnp.int32, (LANES,), 0)
        for i in range(CW // LANES):
            cv[pl.ds(i * LANES, LANES)] = jnp.where(lane == 0, nchunks, 0)
        pltpu.sync_copy(cv, cnts.at[pl.ds(pl.multiple_of(wid * CW, 8), CW)])

    return part


@functools.lru_cache(maxsize=None)
def _layer_fn(E, with_ub):
    CAP = _round_up(E, STAGE_E)
    out_type = [jax.ShapeDtypeStruct((NPAD, D), jnp.float32)]
    if with_ub:
        out_type.append(jax.ShapeDtypeStruct((NW * UBW,), jnp.float32))
    QW = 32
    NQF = (R + 8) * QW
    NB = 2 if with_ub else 3
    scratch = [
        pltpu.VMEM((NB, C, D), jnp.float32),
        pltpu.VMEM((STAGE_E,), jnp.int32),
        pltpu.VMEM((STAGE_E,), jnp.float32),
        pltpu.VMEM((STAGE_E,), jnp.int32),
        pltpu.VMEM((NQF,), jnp.float32),
        pltpu.VMEM((NQF,), jnp.float32),
        pltpu.VMEM((NQF,), jnp.float32),
        pltpu.VMEM((NQF,), jnp.float32),
        pltpu.VMEM((CW,), jnp.int32),
        pltpu.SemaphoreType.DMA((NB,)),
    ]
    if with_ub:
        scratch.insert(4, pltpu.VMEM((STAGE_E,), jnp.float32))
        scratch.insert(9, pltpu.VMEM(((R + 8) * LANES,), jnp.float32))
        scratch.insert(10, pltpu.VMEM((UBW,), jnp.float32))

    @functools.partial(
        pl.kernel,
        out_type=tuple(out_type) if with_ub else out_type[0],
        mesh=_mesh(),
        compiler_params=pltpu.CompilerParams(needs_layout_passes=False, disable_bounds_checks=True),
        scratch_types=scratch,
    )
    def layer(hp_h, srcL, wL, dlL, pwL, cnts, *rest):
        if with_ub:
            (neigh_o, ub_o, rows, sstg, wstg, dstg, pwstg, nq0, nq1, nq2, nq3,
             ubL, ubc, cv, gsem) = rest
        else:
            (neigh_o, rows, sstg, wstg, dstg, nq0, nq1, nq2, nq3,
             cv, gsem) = rest
            ub_o = ubL = ubc = pwstg = None
        nqs = (nq0, nq1, nq2, nq3)
        wid = _wid()
        lbase = wid * CAP

        pltpu.sync_copy(cnts.at[pl.ds(pl.multiple_of(wid * CW, 8), CW)], cv)
        nc = cv[pl.ds(0, LANES)][0]

        minf = jnp.full((LANES,), NEG_INF, jnp.float32)
        zf = jnp.zeros((LANES,), jnp.float32)

        def initrow(i, _):
            o = i * LANES
            for q in range(4):
                nqs[q][pl.ds(o, LANES)] = minf
            if with_ub:
                ubL[pl.ds(o, LANES)] = zf
            return 0

        lax.fori_loop(0, NQF // LANES, initrow, 0)

        nstages = (nc + STAGE_C - 1) // STAGE_C

        def stage(s, _):
            eb = pl.multiple_of(lbase + s * STAGE_E, 8)
            pltpu.sync_copy(srcL.at[pl.ds(eb, STAGE_E)], sstg)
            pltpu.sync_copy(wL.at[pl.ds(eb, STAGE_E)], wstg)
            pltpu.sync_copy(dlL.at[pl.ds(eb, STAGE_E)], dstg)
            if with_ub:
                pltpu.sync_copy(pwL.at[pl.ds(eb, STAGE_E)], pwstg)
            tc = jnp.minimum(STAGE_C, nc - s * STAGE_C)

            pltpu.async_copy(hp_h.at[sstg.at[pl.ds(pl.multiple_of(0, 8), C)]], rows.at[0], gsem.at[0])
            if NB > 2:
                @pl.when(1 < tc)
                def _():
                    pltpu.async_copy(
                        hp_h.at[sstg.at[pl.ds(pl.multiple_of(C, 8), C)]],
                        rows.at[1], gsem.at[1])

            def chunk(k, _):
                kk = k % NB
                nk = (k + NB - 1) % NB

                @pl.when(k + NB - 1 < tc)
                def _():
                    pltpu.async_copy(hp_h.at[sstg.at[pl.ds(pl.multiple_of((k + NB - 1) * C, 8), C)]],
                                     rows.at[nk], gsem.at[nk])

                pltpu.make_async_copy(hp_h.at[sstg.at[pl.ds(pl.multiple_of(k * C, 8), C)]],
                                      rows.at[kk], gsem.at[kk]).wait()

                def group(g, _):
                    eg = k * C + g * LANES
                    wv = wstg[pl.ds(eg, LANES)]
                    dv = dstg[pl.ds(eg, LANES)] * QW
                    if with_ub:
                        pwv = pwstg[pl.ds(eg, LANES)]
                    wjs = [wv[j] for j in range(LANES)]
                    ojs = [dv[j] for j in range(LANES)]
                    for j in range(LANES):
                        wj = wjs[j]
                        o = ojs[j]
                        jj = g * LANES + j
                        blocks = [(q, u) for q in range(4)
                                  for u in range(QW // LANES)]
                        segs = [rows[kk, jj, pl.ds(q * QW + u * LANES, LANES)]
                                for q, u in blocks]
                        curs = [nqs[q][pl.ds(o + u * LANES, LANES)]
                                for q, u in blocks]
                        news = [jnp.maximum(c, sg * wj)
                                for c, sg in zip(curs, segs)]
                        for (q, u), nv in zip(blocks, news):
                            nqs[q][pl.ds(o + u * LANES, LANES)] = nv
                        if with_ub:
                            ubL[pl.ds(o // 2, LANES)] = \
                                ubL[pl.ds(o // 2, LANES)] + pwv[j]
                    return 0

                lax.fori_loop(0, C // LANES, group, 0)
                return 0

            lax.fori_loop(0, tc, chunk, 0)
            return 0

        lax.fori_loop(0, nstages, stage, 0)

        rowbase = pl.multiple_of(wid * R, 8)
        for b, nrow in ((0, C), (1, C), (2, R - 2 * C)):
            def mrow(r, _):
                for q in range(4):
                    for u in range(QW // LANES):
                        f = q * QW + u * LANES
                        rows[0, r, pl.ds(f, LANES)] = \
                            nqs[q][pl.ds((b * C + r) * QW + u * LANES, LANES)]
                return 0

            lax.fori_loop(0, nrow, mrow, 0)
            pltpu.sync_copy(
                rows.at[0, pl.ds(0, nrow)],
                neigh_o.at[pl.ds(rowbase + b * C, nrow)])
        if with_ub:
            lane = lax.broadcasted_iota(jnp.int32, (LANES,), 0)
            zl = jnp.zeros((LANES,), jnp.int32)

            def compact(g, _):
                vals = plsc.load_gather(ubL, [(g * LANES + lane) * LANES + zl])
                ubc[pl.ds(g * LANES, LANES)] = vals
                return 0

            lax.fori_loop(0, R // LANES, compact, 0)
            pltpu.sync_copy(ubc, ub_o.at[pl.ds(pl.multiple_of(wid * UBW, 8), UBW)])

    return layer


def _dotT(a, b):
    return lax.dot_general(a, b, (((1,), (1,)), ((), ())),
                           preferred_element_type=jnp.float32)


TB = 2000


def _rowblk(width):
    return pl.BlockSpec((TB, width), lambda g: (g, 0))


def _full(shape):
    return pl.BlockSpec(shape, lambda g: tuple(0 for _ in shape))


def _fix_neigh(v):
    return jnp.where(v == NEG_INF, 0.0, v)


def _tc_pre_body(x_ref, wp_ref, bp_ref, o_ref):
    o_ref[...] = _dotT(x_ref[...], wp_ref[...]) + bp_ref[...]


def _tc_pre(x, wp, bp):
    return pl.pallas_call(
        _tc_pre_body,
        grid=(N // TB,),
        in_specs=[_rowblk(D), _full((D, D)), _full((1, D))],
        out_specs=_rowblk(D),
        out_shape=jax.ShapeDtypeStruct((N, D), jnp.float32),
    )(x, wp, bp.reshape(1, D))


def _tc_mid_body(h_ref, n_ref, wn_ref, bn_ref, wp_ref, bp_ref, h1_ref, hp_ref):
    h = h_ref[...]
    nf = _fix_neigh(n_ref[...])
    wn = wn_ref[...]
    h1 = _dotT(h, wn[:, :D]) + _dotT(nf, wn[:, D:]) + bn_ref[...]
    h1 = jnp.maximum(h1, 0.0)
    h1_ref[...] = h1
    hp_ref[...] = _dotT(h1, wp_ref[...]) + bp_ref[...]


def _tc_mid(h, neigh_raw, wn, bn, wp, bp):
    return pl.pallas_call(
        _tc_mid_body,
        grid=(N // TB,),
        in_specs=[_rowblk(D), _rowblk(D), _full((D, 2 * D)), _full((1, D)),
                  _full((D, D)), _full((1, D))],
        out_specs=(_rowblk(D), _rowblk(D)),
        out_shape=(jax.ShapeDtypeStruct((N, D), jnp.float32),
                   jax.ShapeDtypeStruct((N, D), jnp.float32)),
    )(h, neigh_raw, wn, bn.reshape(1, D), wp, bp.reshape(1, D))


def _tc_fin_body(h_ref, n_ref, x_ref, dub_ref, wn_ref, bn_ref, o_ref):
    h = h_ref[...]
    nf = _fix_neigh(n_ref[...])
    wn = wn_ref[...]
    s = _dotT(h, wn[:, :D]) + _dotT(nf, wn[:, D:]) + bn_ref[...]
    h3 = jnp.maximum(s, 0.0)
    now = x_ref[...][:, D - 1:D]
    ub = jnp.clip(now + dub_ref[...], 0.0, 1.0)
    o_ref[...] = jnp.minimum(now + h3, ub)


def _tc_fin(h2, neigh_raw, x, dub, wn, bn):
    return pl.pallas_call(
        _tc_fin_body,
        grid=(N // TB,),
        in_specs=[_rowblk(D), _rowblk(D), _rowblk(D), _rowblk(1),
                  _full((1, 2 * D)), _full((1, 1))],
        out_specs=_rowblk(1),
        out_shape=jax.ShapeDtypeStruct((N, 1), jnp.float32),
    )(h2, neigh_raw, x, dub, wn, bn.reshape(1, 1))


def kernel(x, edge_index, edge_weight, params):
    E = edge_index.shape[1]
    src = edge_index[0].astype(jnp.int32)
    dst = edge_index[1].astype(jnp.int32)
    ep = src | (dst << PACKB)
    w = edge_weight.astype(jnp.float32)
    prv = jnp.pad(x[:, D - 2], (0, NPAD - N))

    srcL, wL, dlL, pwL, cnts = _partition_fn(E)(ep, w, prv)

    hp0 = _tc_pre(x, params["W_pool_0"], params["b_pool_0"])
    neigh0, ub = _layer_fn(E, True)(hp0, srcL, wL, dlL, pwL, cnts)
    h1, hp1 = _tc_mid(x, neigh0, params["W_neigh_0"], params["b_neigh_0"],
                      params["W_pool_1"], params["b_pool_1"])
    neigh1 = _layer_fn(E, False)(hp1, srcL, wL, dlL, pwL, cnts)
    h2, hp2 = _tc_mid(h1, neigh1, params["W_neigh_1"], params["b_neigh_1"],
                      params["W_pool_2"], params["b_pool_2"])
    neigh2 = _layer_fn(E, False)(hp2, srcL, wL, dlL, pwL, cnts)
    dub = ub.reshape(NW, UBW)[:, :R].reshape(NPAD)[:N].reshape(N, 1)
    out = _tc_fin(h2, neigh2, x, dub, params["W_neigh_2"], params["b_neigh_2"])
    return out[:, 0]
